# Initial kernel scaffold; baseline (speedup 1.0000x reference)
import jax
import jax.numpy as jnp
from jax.experimental import pallas as pl


def kernel(query, keys_0, values_0, salience_0, keys_1, values_1, salience_1, keys_2, values_2, salience_2):
    raise NotImplementedError("write your pallas kernel here")


# speedup vs baseline: 2.9907x; 2.9907x over previous
import math

import jax
import jax.numpy as jnp
from jax.experimental import pallas as pl
from jax.experimental.pallas import tpu as pltpu

D_MODEL = 2048
NUM_LVL = 3
SEG = 16
S_TOTAL = NUM_LVL * SEG
INV_SQRT_D = 1.0 / math.sqrt(D_MODEL)
LEVEL_W = 1.0 / NUM_LVL
BLK = 512


def _attn_kernel(q_ref, kt_ref, v_ref, sal_ref, o_ref):
    q = q_ref[...]
    s = jnp.dot(q, kt_ref[...], preferred_element_type=jnp.float32)
    s = s * INV_SQRT_D + sal_ref[...]
    m = jnp.max(s, axis=1, keepdims=True)
    e = jnp.exp(s - m)
    i = jax.lax.broadcasted_iota(jnp.int32, (S_TOTAL, S_TOTAL), 0) // SEG
    j = jax.lax.broadcasted_iota(jnp.int32, (S_TOTAL, S_TOTAL), 1) // SEG
    seg = (i == j).astype(jnp.float32)
    z = jax.lax.dot_general(
        e, seg, (((1,), (0,)), ((), ())),
        precision=jax.lax.Precision.HIGHEST,
        preferred_element_type=jnp.float32,
    )
    a = (e / z) * LEVEL_W
    o_ref[...] = jnp.dot(a, v_ref[...], preferred_element_type=jnp.float32)


@jax.jit
def kernel(query, keys_0, values_0, salience_0, keys_1, values_1, salience_1,
           keys_2, values_2, salience_2):
    B, T, D = query.shape
    q2 = query.reshape(B * T, D)
    kt = jnp.concatenate([keys_0, keys_1, keys_2], axis=0).T
    v = jnp.concatenate([values_0, values_1, values_2], axis=0)
    sal = jnp.concatenate([salience_0, salience_1, salience_2]).reshape(1, S_TOTAL)
    grid = ((B * T) // BLK,)
    out = pl.pallas_call(
        _attn_kernel,
        grid=grid,
        in_specs=[
            pl.BlockSpec((BLK, D), lambda i: (i, 0)),
            pl.BlockSpec((D, S_TOTAL), lambda i: (0, 0)),
            pl.BlockSpec((S_TOTAL, D), lambda i: (0, 0)),
            pl.BlockSpec((1, S_TOTAL), lambda i: (0, 0)),
        ],
        out_specs=pl.BlockSpec((BLK, D), lambda i: (i, 0)),
        out_shape=jax.ShapeDtypeStruct((B * T, D), jnp.float32),
        compiler_params=pltpu.CompilerParams(dimension_semantics=("parallel",)),
    )(q2, kt, v, sal)
    return out.reshape(B, T, D)
